# baseline (speedup 1.0000x reference)
import functools

import jax
import jax.numpy as jnp
from jax import lax
from jax.experimental import pallas as pl
from jax.experimental.pallas import tpu as pltpu
from jax.experimental.pallas import tpu_sc as plsc

N_ATOMS_TOTAL = 1048576
N_GRAPHS = 16384
ATOMS_PER_GRAPH = 64
N_ELEMENTS = 119
TABLE_PAD = 128

NUM_CORES = 2
NUM_SUBCORES = 16
NUM_WORKERS = NUM_CORES * NUM_SUBCORES
LANES = 16

ATOMS_PER_WORKER = N_ATOMS_TOTAL // NUM_WORKERS
GRAPHS_PER_WORKER = N_GRAPHS // NUM_WORKERS
GROUPS_PER_WORKER = GRAPHS_PER_WORKER // LANES


N_CHUNKS = 4
ATOMS_PER_CHUNK = ATOMS_PER_WORKER // N_CHUNKS
GROUPS_PER_CHUNK = GROUPS_PER_WORKER // N_CHUNKS


def _sc_body(table_hbm, atoms_hbm, out_hbm, table_v, traw_v, atoms_v, acc_v,
             *sems):
    cid = lax.axis_index("c")
    sid = lax.axis_index("s")
    wid = sid * NUM_CORES + cid
    base = wid * ATOMS_PER_WORKER

    cps = [
        pltpu.async_copy(
            atoms_hbm.at[pl.ds(base + c * ATOMS_PER_CHUNK, ATOMS_PER_CHUNK)],
            atoms_v.at[pl.ds(c * ATOMS_PER_CHUNK, ATOMS_PER_CHUNK)],
            sems[c])
        for c in range(N_CHUNKS)
    ]

    lane = lax.iota(jnp.int32, LANES)
    lane_base = lane * ATOMS_PER_GRAPH

    pltpu.sync_copy(table_hbm, traw_v.at[pl.ds(0, N_ELEMENTS)])
    for c in range((N_ELEMENTS + LANES - 1) // LANES):
        chunk = traw_v[pl.ds(c * LANES, LANES)]
        for k in range(min(LANES, N_ELEMENTS - c * LANES)):
            e = c * LANES + k
            table_v[pl.ds(e * LANES, LANES)] = chunk.at[
                jnp.full((LANES,), k, jnp.int32)].get(mode="promise_in_bounds")

    def group_body(g, _):
        idx0 = lane_base + g * (LANES * ATOMS_PER_GRAPH)

        accs = [jnp.zeros((LANES,), jnp.float32) for _ in range(4)]
        for j in range(ATOMS_PER_GRAPH):
            rot = (lane + j) & (ATOMS_PER_GRAPH - 1)
            ids = plsc.load_gather(atoms_v, [idx0 + rot])
            accs[j % 4] = accs[j % 4] + plsc.load_gather(
                table_v, [(ids << 4) + lane])
        acc_v[pl.ds(g * LANES, LANES)] = (accs[0] + accs[1]) + (accs[2] + accs[3])
        return 0

    for c in range(N_CHUNKS):
        cps[c].wait()

        @plsc.parallel_loop(c * GROUPS_PER_CHUNK, (c + 1) * GROUPS_PER_CHUNK)
        def _(g):
            group_body(g, 0)

    pltpu.sync_copy(acc_v,
                    out_hbm.at[pl.ds(wid * GRAPHS_PER_WORKER, GRAPHS_PER_WORKER)])


@functools.partial(
    pl.kernel,
    out_type=jax.ShapeDtypeStruct((N_GRAPHS,), jnp.float32),
    mesh=plsc.VectorSubcoreMesh(
        core_axis_name="c", subcore_axis_name="s",
        num_cores=NUM_CORES, num_subcores=NUM_SUBCORES),
    scratch_types=[
        pltpu.VMEM((N_ELEMENTS * LANES,), jnp.float32),
        pltpu.VMEM((TABLE_PAD,), jnp.float32),
        pltpu.VMEM((ATOMS_PER_WORKER,), jnp.int32),
        pltpu.VMEM((GRAPHS_PER_WORKER,), jnp.float32),
        pltpu.SemaphoreType.DMA,
        pltpu.SemaphoreType.DMA,
        pltpu.SemaphoreType.DMA,
        pltpu.SemaphoreType.DMA,
    ],
    compiler_params=pltpu.CompilerParams(needs_layout_passes=False),
)
def _pooled_sum(table_hbm, atoms_hbm, out_hbm, table_v, traw_v, atoms_v, acc_v,
                *sems):
    _sc_body(table_hbm, atoms_hbm, out_hbm, table_v, traw_v, atoms_v, acc_v,
             *sems)


def kernel(property_per_element, atom_features, n_atoms):
    del n_atoms
    pooled = _pooled_sum(property_per_element, atom_features)
    return pooled.reshape(-1, 1)


# speedup vs baseline: 1.2631x; 1.2631x over previous
import functools

import jax
import jax.numpy as jnp
from jax import lax
from jax.experimental import pallas as pl
from jax.experimental.pallas import tpu as pltpu
from jax.experimental.pallas import tpu_sc as plsc

N_ATOMS_TOTAL = 1048576
N_GRAPHS = 16384
ATOMS_PER_GRAPH = 64
N_ELEMENTS = 119
TABLE_PAD = 128

NUM_CORES = 2
NUM_SUBCORES = 16
NUM_WORKERS = NUM_CORES * NUM_SUBCORES
LANES = 16

ATOMS_PER_WORKER = N_ATOMS_TOTAL // NUM_WORKERS
GRAPHS_PER_WORKER = N_GRAPHS // NUM_WORKERS
GROUPS_PER_WORKER = GRAPHS_PER_WORKER // LANES


N_CHUNKS = 4
ATOMS_PER_CHUNK = ATOMS_PER_WORKER // N_CHUNKS
GROUPS_PER_CHUNK = GROUPS_PER_WORKER // N_CHUNKS


def _sc_body(table_hbm, atoms_hbm, out_hbm, table_v, traw_v, atoms_v, acc_v,
             *sems):
    cid = lax.axis_index("c")
    sid = lax.axis_index("s")
    wid = sid * NUM_CORES + cid
    base = wid * ATOMS_PER_WORKER

    cps = [
        pltpu.async_copy(
            atoms_hbm.at[pl.ds(base + c * ATOMS_PER_CHUNK, ATOMS_PER_CHUNK)],
            atoms_v.at[pl.ds(c * ATOMS_PER_CHUNK, ATOMS_PER_CHUNK)],
            sems[c])
        for c in range(N_CHUNKS)
    ]

    lane = lax.iota(jnp.int32, LANES)
    lane_base = lane * ATOMS_PER_GRAPH

    pltpu.sync_copy(table_hbm, traw_v.at[pl.ds(0, N_ELEMENTS)])
    for c in range((N_ELEMENTS + LANES - 1) // LANES):
        chunk = traw_v[pl.ds(c * LANES, LANES)]
        for k in range(min(LANES, N_ELEMENTS - c * LANES)):
            e = c * LANES + k
            table_v[pl.ds(e * LANES, LANES)] = chunk.at[
                jnp.full((LANES,), k, jnp.int32)].get(mode="promise_in_bounds")

    def group_body(g, _):
        idx0 = lane_base + g * (LANES * ATOMS_PER_GRAPH)

        accs = [jnp.zeros((LANES,), jnp.float32) for _ in range(4)]
        for j in range(ATOMS_PER_GRAPH):
            rot = (lane + j) & (ATOMS_PER_GRAPH - 1)
            ids = plsc.load_gather(atoms_v, [idx0 + rot])
            accs[j % 4] = accs[j % 4] + plsc.load_gather(
                table_v, [(ids << 4) + lane])
        acc_v[pl.ds(g * LANES, LANES)] = (accs[0] + accs[1]) + (accs[2] + accs[3])
        return 0

    for c in range(N_CHUNKS):
        cps[c].wait()
        lax.fori_loop(c * GROUPS_PER_CHUNK, (c + 1) * GROUPS_PER_CHUNK,
                      group_body, 0)

    pltpu.sync_copy(acc_v,
                    out_hbm.at[pl.ds(wid * GRAPHS_PER_WORKER, GRAPHS_PER_WORKER)])


@functools.partial(
    pl.kernel,
    out_type=jax.ShapeDtypeStruct((N_GRAPHS,), jnp.float32),
    mesh=plsc.VectorSubcoreMesh(
        core_axis_name="c", subcore_axis_name="s",
        num_cores=NUM_CORES, num_subcores=NUM_SUBCORES),
    scratch_types=[
        pltpu.VMEM((N_ELEMENTS * LANES,), jnp.float32),
        pltpu.VMEM((TABLE_PAD,), jnp.float32),
        pltpu.VMEM((ATOMS_PER_WORKER,), jnp.int32),
        pltpu.VMEM((GRAPHS_PER_WORKER,), jnp.float32),
        pltpu.SemaphoreType.DMA,
        pltpu.SemaphoreType.DMA,
        pltpu.SemaphoreType.DMA,
        pltpu.SemaphoreType.DMA,
    ],
    compiler_params=pltpu.CompilerParams(needs_layout_passes=False),
)
def _pooled_sum(table_hbm, atoms_hbm, out_hbm, table_v, traw_v, atoms_v, acc_v,
                *sems):
    _sc_body(table_hbm, atoms_hbm, out_hbm, table_v, traw_v, atoms_v, acc_v,
             *sems)


def kernel(property_per_element, atom_features, n_atoms):
    del n_atoms
    pooled = _pooled_sum(property_per_element, atom_features)
    return pooled.reshape(-1, 1)
